# baseline (speedup 1.0000x reference)
import functools

import jax
import jax.numpy as jnp
from jax import lax
from jax.experimental import pallas as pl
from jax.experimental.pallas import tpu as pltpu
from jax.experimental.pallas import tpu_sc as plsc

B = 4096
D = 128
H = 512
NCHUNK = 2
CHUNK = B // NCHUNK


def _gather_sc(user_ids, item_ids, user_table, item_table):
    n = user_ids.shape[0]
    info = plsc.get_sparse_core_info()
    nw = info.num_cores * info.num_subcores
    bpw = n // nw
    mesh = plsc.VectorSubcoreMesh(core_axis_name="c", subcore_axis_name="s")

    @functools.partial(
        pl.kernel,
        mesh=mesh,
        out_type=(jax.ShapeDtypeStruct((n, D), jnp.float32),
                  jax.ShapeDtypeStruct((n, D), jnp.float32)),
        scratch_types=[
            pltpu.VMEM((bpw,), jnp.int32),
            pltpu.VMEM((bpw,), jnp.int32),
            pltpu.VMEM((bpw, D), jnp.float32),
            pltpu.VMEM((bpw, D), jnp.float32),
            pltpu.SemaphoreType.DMA,
            pltpu.SemaphoreType.DMA,
        ],
    )
    def k(uids_hbm, iids_hbm, ut_hbm, it_hbm, uout_hbm, vout_hbm,
          uidx, iidx, urows, vrows, sem_u, sem_v):
        wid = lax.axis_index("s") * info.num_cores + lax.axis_index("c")
        base = wid * bpw
        cu_idx = pltpu.async_copy(uids_hbm.at[pl.ds(base, bpw)], uidx, sem_u)
        cv_idx = pltpu.async_copy(iids_hbm.at[pl.ds(base, bpw)], iidx, sem_v)
        cu_idx.wait()
        cu = pltpu.async_copy(ut_hbm.at[uidx], urows, sem_u)
        cv_idx.wait()
        cv = pltpu.async_copy(it_hbm.at[iidx], vrows, sem_v)
        cu.wait()
        cu_out = pltpu.async_copy(urows, uout_hbm.at[pl.ds(base, bpw)], sem_u)
        cv.wait()
        cv_out = pltpu.async_copy(vrows, vout_hbm.at[pl.ds(base, bpw)], sem_v)
        cu_out.wait()
        cv_out.wait()

    return k(user_ids, item_ids, user_table, item_table)


def _mlp_body(u_ref, v_ref, w1_ref, b1_ref, w2_ref, b2_ref, *refs):
    *prev, out_ref = refs
    del prev
    x = (u_ref[...] * v_ref[...]).astype(jnp.bfloat16)
    h = jnp.dot(x, w1_ref[...], preferred_element_type=jnp.float32)
    h = jnp.maximum(h + b1_ref[...], 0.0).astype(jnp.bfloat16)
    out = jnp.dot(h, w2_ref[...], preferred_element_type=jnp.float32)
    out_ref[...] = out + b2_ref[...]


def _mlp_tc_chunk(u, v, W1b, b1, W2b, b2, prev, chunk_idx):
    in_specs = [
        pl.BlockSpec((CHUNK, D), lambda i: (0, 0)),
        pl.BlockSpec((CHUNK, D), lambda i: (0, 0)),
        pl.BlockSpec((D, H), lambda i: (0, 0)),
        pl.BlockSpec((1, H), lambda i: (0, 0)),
        pl.BlockSpec((H, H), lambda i: (0, 0)),
        pl.BlockSpec((1, H), lambda i: (0, 0)),
    ]
    args = [u, v, W1b, b1, W2b, b2]
    aliases = {}
    if prev is not None:
        in_specs.append(pl.BlockSpec(memory_space=pl.ANY))
        args.append(prev)
        aliases = {6: 0}
    return pl.pallas_call(
        _mlp_body,
        grid=(1,),
        in_specs=in_specs,
        out_specs=pl.BlockSpec((CHUNK, H), lambda i, c=chunk_idx: (c, 0)),
        out_shape=jax.ShapeDtypeStruct((B, H), jnp.float32),
        input_output_aliases=aliases,
    )(*args)


def kernel(user_ids, item_ids, user_table, item_table, W1, b1, W2, b2):
    uids = user_ids.astype(jnp.int32)
    iids = item_ids.astype(jnp.int32)
    W1b = W1.astype(jnp.bfloat16)
    W2b = W2.astype(jnp.bfloat16)
    b1r = b1.reshape(1, H)
    b2r = b2.reshape(1, H)

    chunks = [
        _gather_sc(uids[c * CHUNK:(c + 1) * CHUNK],
                   iids[c * CHUNK:(c + 1) * CHUNK],
                   user_table, item_table)
        for c in range(NCHUNK)
    ]
    out = None
    for c, (u, v) in enumerate(chunks):
        out = _mlp_tc_chunk(u, v, W1b, b1r, W2b, b2r, out, c)
    return out


# speedup vs baseline: 1.1045x; 1.1045x over previous
import functools

import jax
import jax.numpy as jnp
from jax import lax
from jax.experimental import pallas as pl
from jax.experimental.pallas import tpu as pltpu
from jax.experimental.pallas import tpu_sc as plsc

B = 4096
D = 128
H = 512


def _gather_sc(user_ids, item_ids, user_table, item_table):
    info = plsc.get_sparse_core_info()
    nw = info.num_cores * info.num_subcores
    bpw = B // nw
    sub = bpw // 2
    mesh = plsc.VectorSubcoreMesh(core_axis_name="c", subcore_axis_name="s")

    @functools.partial(
        pl.kernel,
        mesh=mesh,
        out_type=(jax.ShapeDtypeStruct((B, D), jnp.float32),
                  jax.ShapeDtypeStruct((B, D), jnp.float32)),
        scratch_types=[
            pltpu.VMEM((bpw,), jnp.int32),
            pltpu.VMEM((bpw,), jnp.int32),
            pltpu.VMEM((bpw, D), jnp.float32),
            pltpu.VMEM((bpw, D), jnp.float32),
            pltpu.SemaphoreType.DMA,
            pltpu.SemaphoreType.DMA,
            pltpu.SemaphoreType.DMA,
            pltpu.SemaphoreType.DMA,
        ],
    )
    def k(uids_hbm, iids_hbm, ut_hbm, it_hbm, uout_hbm, vout_hbm,
          uidx, iidx, urows, vrows, sem_a, sem_b, sem_c, sem_d):
        wid = lax.axis_index("s") * info.num_cores + lax.axis_index("c")
        base = wid * bpw
        cu_idx = pltpu.async_copy(uids_hbm.at[pl.ds(base, bpw)], uidx, sem_a)
        cv_idx = pltpu.async_copy(iids_hbm.at[pl.ds(base, bpw)], iidx, sem_b)
        cu_idx.wait()
        gu0 = pltpu.async_copy(ut_hbm.at[uidx.at[pl.ds(0, sub)]],
                               urows.at[pl.ds(0, sub)], sem_a)
        gu1 = pltpu.async_copy(ut_hbm.at[uidx.at[pl.ds(sub, sub)]],
                               urows.at[pl.ds(sub, sub)], sem_c)
        cv_idx.wait()
        gv0 = pltpu.async_copy(it_hbm.at[iidx.at[pl.ds(0, sub)]],
                               vrows.at[pl.ds(0, sub)], sem_b)
        gv1 = pltpu.async_copy(it_hbm.at[iidx.at[pl.ds(sub, sub)]],
                               vrows.at[pl.ds(sub, sub)], sem_d)
        gu0.wait()
        wu0 = pltpu.async_copy(urows.at[pl.ds(0, sub)],
                               uout_hbm.at[pl.ds(base, sub)], sem_a)
        gv0.wait()
        wv0 = pltpu.async_copy(vrows.at[pl.ds(0, sub)],
                               vout_hbm.at[pl.ds(base, sub)], sem_b)
        gu1.wait()
        wu1 = pltpu.async_copy(urows.at[pl.ds(sub, sub)],
                               uout_hbm.at[pl.ds(base + sub, sub)], sem_c)
        gv1.wait()
        wv1 = pltpu.async_copy(vrows.at[pl.ds(sub, sub)],
                               vout_hbm.at[pl.ds(base + sub, sub)], sem_d)
        wu0.wait()
        wv0.wait()
        wu1.wait()
        wv1.wait()

    return k(user_ids, item_ids, user_table, item_table)


def _mlp_body(u_ref, v_ref, w1_ref, b1_ref, w2_ref, b2_ref, out_ref):
    x = (u_ref[...] * v_ref[...]).astype(jnp.bfloat16)
    h = jnp.dot(x, w1_ref[...], preferred_element_type=jnp.float32)
    h = jnp.maximum(h + b1_ref[...], 0.0).astype(jnp.bfloat16)
    out = jnp.dot(h, w2_ref[...], preferred_element_type=jnp.float32)
    out_ref[...] = out + b2_ref[...]


def _mlp_tc(u, v, W1, b1, W2, b2):
    blk = 2048
    return pl.pallas_call(
        _mlp_body,
        grid=(B // blk,),
        in_specs=[
            pl.BlockSpec((blk, D), lambda i: (i, 0)),
            pl.BlockSpec((blk, D), lambda i: (i, 0)),
            pl.BlockSpec((D, H), lambda i: (0, 0)),
            pl.BlockSpec((1, H), lambda i: (0, 0)),
            pl.BlockSpec((H, H), lambda i: (0, 0)),
            pl.BlockSpec((1, H), lambda i: (0, 0)),
        ],
        out_specs=pl.BlockSpec((blk, H), lambda i: (i, 0)),
        out_shape=jax.ShapeDtypeStruct((B, H), jnp.float32),
    )(u, v, W1.astype(jnp.bfloat16), b1, W2.astype(jnp.bfloat16), b2)


def kernel(user_ids, item_ids, user_table, item_table, W1, b1, W2, b2):
    u, v = _gather_sc(user_ids.astype(jnp.int32), item_ids.astype(jnp.int32),
                      user_table, item_table)
    return _mlp_tc(u, v, W1, b1.reshape(1, H), W2, b2.reshape(1, H))


# speedup vs baseline: 1.1229x; 1.0166x over previous
import functools

import jax
import jax.numpy as jnp
from jax import lax
from jax.experimental import pallas as pl
from jax.experimental.pallas import tpu as pltpu
from jax.experimental.pallas import tpu_sc as plsc

B = 4096
D = 128
H = 512


def _gather_sc(user_ids, item_ids, user_table, item_table):
    info = plsc.get_sparse_core_info()
    nw = info.num_cores * info.num_subcores
    bpw = B // nw
    mesh = plsc.VectorSubcoreMesh(core_axis_name="c", subcore_axis_name="s")

    @functools.partial(
        pl.kernel,
        mesh=mesh,
        out_type=(jax.ShapeDtypeStruct((B, D), jnp.float32),
                  jax.ShapeDtypeStruct((B, D), jnp.float32)),
        scratch_types=[
            pltpu.VMEM((bpw,), jnp.int32),
            pltpu.VMEM((bpw,), jnp.int32),
            pltpu.VMEM((bpw, D), jnp.float32),
            pltpu.VMEM((bpw, D), jnp.float32),
            pltpu.SemaphoreType.DMA,
            pltpu.SemaphoreType.DMA,
            pltpu.SemaphoreType.DMA,
            pltpu.SemaphoreType.DMA,
        ],
    )
    def k(uids_hbm, iids_hbm, ut_hbm, it_hbm, uout_hbm, vout_hbm,
          uidx, iidx, urows, vrows, sem_a, sem_b, sem_c, sem_d):
        wid = lax.axis_index("s") * info.num_cores + lax.axis_index("c")
        base = wid * bpw
        cu_idx = pltpu.async_copy(uids_hbm.at[pl.ds(base, bpw)], uidx, sem_a)
        cv_idx = pltpu.async_copy(iids_hbm.at[pl.ds(base, bpw)], iidx, sem_b)
        cu_idx.wait()
        gu = pltpu.async_copy(ut_hbm.at[uidx], urows, sem_a)
        cv_idx.wait()
        gv = pltpu.async_copy(it_hbm.at[iidx], vrows, sem_b)
        gu.wait()
        wu = pltpu.async_copy(urows, uout_hbm.at[pl.ds(base, bpw)], sem_c)
        gv.wait()
        wv = pltpu.async_copy(vrows, vout_hbm.at[pl.ds(base, bpw)], sem_d)
        wu.wait()
        wv.wait()

    return k(user_ids, item_ids, user_table, item_table)


def _mlp_body(u_ref, v_ref, w1_ref, b1_ref, w2_ref, b2_ref, out_ref):
    x = (u_ref[...] * v_ref[...]).astype(jnp.bfloat16)
    h = jnp.dot(x, w1_ref[...], preferred_element_type=jnp.float32)
    h = jnp.maximum(h + b1_ref[...], 0.0).astype(jnp.bfloat16)
    out = jnp.dot(h, w2_ref[...], preferred_element_type=jnp.float32)
    out_ref[...] = out + b2_ref[...]


def _mlp_tc(u, v, W1, b1, W2, b2):
    blk = 2048
    return pl.pallas_call(
        _mlp_body,
        grid=(B // blk,),
        in_specs=[
            pl.BlockSpec((blk, D), lambda i: (i, 0)),
            pl.BlockSpec((blk, D), lambda i: (i, 0)),
            pl.BlockSpec((D, H), lambda i: (0, 0)),
            pl.BlockSpec((1, H), lambda i: (0, 0)),
            pl.BlockSpec((H, H), lambda i: (0, 0)),
            pl.BlockSpec((1, H), lambda i: (0, 0)),
        ],
        out_specs=pl.BlockSpec((blk, H), lambda i: (i, 0)),
        out_shape=jax.ShapeDtypeStruct((B, H), jnp.float32),
    )(u, v, W1.astype(jnp.bfloat16), b1, W2.astype(jnp.bfloat16), b2)


def kernel(user_ids, item_ids, user_table, item_table, W1, b1, W2, b2):
    u, v = _gather_sc(user_ids, item_ids, user_table, item_table)
    return _mlp_tc(u, v, W1, b1.reshape(1, H), W2, b2.reshape(1, H))
